# baseline (speedup 1.0000x reference)
import functools

import jax
import jax.numpy as jnp
from jax import lax
from jax.experimental import pallas as pl
from jax.experimental.pallas import tpu as pltpu
from jax.experimental.pallas import tpu_sc as plsc

E = 3_200_000
EMB = 16
OUT_D = 32
NC = 2
NS = 16
L = 16
NW = NC * NS
E_PER_W = E // NW
CHUNK = 2000
NCHUNK = E_PER_W // CHUNK
GROUPS = CHUNK // L

_mesh = plsc.VectorSubcoreMesh(core_axis_name="c", subcore_axis_name="s")


@functools.partial(
    pl.kernel,
    mesh=_mesh,
    compiler_params=pltpu.CompilerParams(
        needs_layout_passes=False, use_tc_tiling_on_sc=False
    ),
    out_type=jax.ShapeDtypeStruct((E * OUT_D,), jnp.float32),
    scratch_types=[
        pltpu.VMEM((4, EMB), jnp.float32),
        pltpu.VMEM((4, EMB), jnp.float32),
        pltpu.VMEM((CHUNK, 2), jnp.int32),
        pltpu.VMEM((CHUNK * OUT_D,), jnp.float32),
    ],
)
def _edge_encode(edge_hbm, w0_hbm, w1_hbm, out_hbm, w0_v, w1_v, idx_v, out_v):
    wid = lax.axis_index("s") * NC + lax.axis_index("c")
    base = wid * E_PER_W
    pltpu.sync_copy(w0_hbm, w0_v)
    pltpu.sync_copy(w1_hbm, w1_v)

    iota = lax.iota(jnp.int32, L)
    zeros = jnp.zeros((L,), jnp.int32)
    ones = jnp.ones((L,), jnp.int32)

    def chunk_body(ci, carry):
        start = base + ci * CHUNK
        pltpu.sync_copy(edge_hbm.at[pl.ds(start, CHUNK)], idx_v)

        def grp(gi, c2):
            e_loc = iota + gi * L
            a0 = plsc.load_gather(idx_v, [e_loc, zeros])
            a1 = plsc.load_gather(idx_v, [e_loc, ones])
            e32 = e_loc * OUT_D
            for c in range(EMB):
                cc = jnp.full((L,), c, jnp.int32)
                v0 = plsc.load_gather(w0_v, [a0, cc])
                plsc.store_scatter(out_v, [e32 + c], v0)
                v1 = plsc.load_gather(w1_v, [a1, cc])
                plsc.store_scatter(out_v, [e32 + (EMB + c)], v1)
            return c2

        lax.fori_loop(0, GROUPS, grp, 0)
        pltpu.sync_copy(out_v, out_hbm.at[pl.ds(start * OUT_D, CHUNK * OUT_D)])
        return carry

    lax.fori_loop(0, NCHUNK, chunk_body, 0)


def kernel(edge_attr, W0, W1):
    return _edge_encode(edge_attr, W0, W1).reshape(E, OUT_D)


# speedup vs baseline: 5.0880x; 5.0880x over previous
import functools

import jax
import jax.numpy as jnp
from jax import lax
from jax.experimental import pallas as pl
from jax.experimental.pallas import tpu as pltpu
from jax.experimental.pallas import tpu_sc as plsc

E = 3_200_000
EMB = 16
OUT_D = 32
NC = 2
NS = 16
L = 16
NW = NC * NS
EB = E // 128
BPC = 8
CHUNK_E = BPC * 128
NCHUNKS = EB // BPC
CPW = -(-NCHUNKS // NW)
GROUPS = CHUNK_E // L

_mesh = plsc.VectorSubcoreMesh(core_axis_name="c", subcore_axis_name="s")


@functools.partial(
    pl.kernel,
    mesh=_mesh,
    compiler_params=pltpu.CompilerParams(
        needs_layout_passes=False, use_tc_tiling_on_sc=False
    ),
    out_type=jax.ShapeDtypeStruct((E * OUT_D,), jnp.float32),
    scratch_types=[
        pltpu.VMEM((4, EMB), jnp.float32),
        pltpu.VMEM((4, EMB), jnp.float32),
        pltpu.VMEM((CHUNK_E * 2,), jnp.int32),
        pltpu.VMEM((CHUNK_E * OUT_D,), jnp.float32),
    ],
)
def _edge_encode(edge_hbm, w0_hbm, w1_hbm, out_hbm, w0_v, w1_v, idx_v, out_v):
    wid = lax.axis_index("s") * NC + lax.axis_index("c")
    pltpu.sync_copy(w0_hbm, w0_v)
    pltpu.sync_copy(w1_hbm, w1_v)

    iota = lax.iota(jnp.int32, L)

    def chunk_body(k, carry):
        ci = k * NW + wid

        @pl.when(ci < NCHUNKS)
        def _():
            t0 = ci * BPC
            pltpu.sync_copy(edge_hbm.at[pl.ds(t0 * 256, BPC * 256)], idx_v)

            def grp(g, c2):
                b = g >> 3
                ll0 = (g & 7) << 4
                in_addr = (b * 256 + ll0) + iota
                a0 = plsc.load_gather(idx_v, [in_addr])
                a1 = plsc.load_gather(idx_v, [in_addr + 128])
                ob = (b * 1024 + ll0) + iota
                for d in range(OUT_D):
                    r, m = d >> 3, d & 7
                    cc = jnp.full((L,), d % EMB, jnp.int32)
                    if d < EMB:
                        v = plsc.load_gather(w0_v, [a0, cc])
                    else:
                        v = plsc.load_gather(w1_v, [a1, cc])
                    plsc.store_scatter(out_v, [ob + (r * BPC * 1024 + m * 128)], v)
                return c2

            lax.fori_loop(0, GROUPS, grp, 0)
            for r in range(4):
                pltpu.sync_copy(
                    out_v.at[pl.ds(r * BPC * 1024, BPC * 1024)],
                    out_hbm.at[pl.ds((r * EB + t0) * 1024, BPC * 1024)],
                )

        return carry

    lax.fori_loop(0, CPW, chunk_body, 0)


def kernel(edge_attr, W0, W1):
    ea_lin = edge_attr.reshape(EB, 128, 2).transpose(0, 2, 1).reshape(E * 2)
    flat = _edge_encode(ea_lin, W0, W1)
    return flat.reshape(4, EB, 8, 128).transpose(1, 3, 0, 2).reshape(E, OUT_D)


# speedup vs baseline: 11.4062x; 2.2418x over previous
import functools

import jax
import jax.numpy as jnp
from jax import lax
from jax.experimental import pallas as pl
from jax.experimental.pallas import tpu as pltpu
from jax.experimental.pallas import tpu_sc as plsc

E = 3_200_000
EMB = 16
OUT_D = 32
NC = 2
NS = 16
L = 16
NW = NC * NS
EB = E // 128
BPC = 8
CHUNK_E = BPC * 128
NCHUNKS = EB // BPC
CPW = -(-NCHUNKS // NW)
GROUPS = CHUNK_E // L

_mesh = plsc.VectorSubcoreMesh(core_axis_name="c", subcore_axis_name="s")


@functools.partial(
    pl.kernel,
    mesh=_mesh,
    compiler_params=pltpu.CompilerParams(
        needs_layout_passes=False, use_tc_tiling_on_sc=False
    ),
    out_type=jax.ShapeDtypeStruct((E * OUT_D,), jnp.float32),
    scratch_types=[
        pltpu.VMEM((4, EMB), jnp.float32),
        pltpu.VMEM((4, EMB), jnp.float32),
        pltpu.VMEM((CHUNK_E * 2,), jnp.int32),
        pltpu.VMEM((CHUNK_E * OUT_D,), jnp.float32),
    ],
)
def _edge_encode(edge_hbm, w0_hbm, w1_hbm, out_hbm, w0_v, w1_v, idx_v, out_v):
    wid = lax.axis_index("s") * NC + lax.axis_index("c")
    pltpu.sync_copy(w0_hbm, w0_v)
    pltpu.sync_copy(w1_hbm, w1_v)

    iota = lax.iota(jnp.int32, L)

    def chunk_body(k, carry):
        ci = k * NW + wid

        @pl.when(ci < NCHUNKS)
        def _():
            t0 = ci * BPC
            pltpu.sync_copy(edge_hbm.at[pl.ds(t0 * 256, BPC * 256)], idx_v)

            @plsc.parallel_loop(0, GROUPS, 1, unroll=2)
            def _grp(g):
                b = g >> 3
                ll0 = (g & 7) << 4
                in_addr = (b * 256 + ll0) + iota
                a0 = plsc.load_gather(idx_v, [in_addr])
                a1 = plsc.load_gather(idx_v, [in_addr + 128])
                ob = (b * 1024 + ll0) + iota
                for d in range(OUT_D):
                    r, m = d >> 3, d & 7
                    cc = jnp.full((L,), d % EMB, jnp.int32)
                    if d < EMB:
                        v = plsc.load_gather(w0_v, [a0, cc])
                    else:
                        v = plsc.load_gather(w1_v, [a1, cc])
                    plsc.store_scatter(out_v, [ob + (r * BPC * 1024 + m * 128)], v)
            for r in range(4):
                pltpu.sync_copy(
                    out_v.at[pl.ds(r * BPC * 1024, BPC * 1024)],
                    out_hbm.at[pl.ds((r * EB + t0) * 1024, BPC * 1024)],
                )

        return carry

    lax.fori_loop(0, CPW, chunk_body, 0)


def kernel(edge_attr, W0, W1):
    ea_lin = edge_attr.reshape(EB, 128, 2).transpose(0, 2, 1).reshape(E * 2)
    flat = _edge_encode(ea_lin, W0, W1)
    return flat.reshape(4, EB, 8, 128).transpose(1, 3, 0, 2).reshape(E, OUT_D)
